# baseline (speedup 1.0000x reference)
import functools

import jax
import jax.numpy as jnp
from jax import lax
from jax.experimental import pallas as pl
from jax.experimental.pallas import tpu as pltpu
from jax.experimental.pallas import tpu_sc as plsc

_NW = 32
_C = 2048
_IGN = 100


@functools.lru_cache(maxsize=None)
def _build(n_nets, n_pins, G, SS):
    mesh = plsc.VectorSubcoreMesh(core_axis_name="c", subcore_axis_name="s")
    ninf = jnp.float32(-jnp.inf)
    pinf = jnp.float32(jnp.inf)

    fp_clamp = n_pins - _C
    assert fp_clamp % 8 == 0 and fp_clamp > 0

    @functools.partial(
        pl.kernel,
        mesh=mesh,
        out_type=jax.ShapeDtypeStruct((_NW * G,), jnp.float32),
        scratch_types=[
            pltpu.VMEM((SS,), jnp.int32),
            pltpu.VMEM((4 * _C,), jnp.int32),
            pltpu.VMEM((4 * _C,), jnp.float32),
            pltpu.VMEM((G + 16,), jnp.float32),
            pltpu.SemaphoreType.DMA,
        ],
        compiler_params=pltpu.CompilerParams(needs_layout_passes=False),
    )
    def k(ns_hbm, fp2_hbm, posi_hbm, out_hbm,
          ns_v, idx_v, v_v, out_v, sem):
        wid = lax.axis_index("s") * 2 + lax.axis_index("c")
        nw0 = wid * G
        pltpu.sync_copy(ns_hbm.at[pl.ds(nw0, SS)], ns_v)
        s0 = ns_v[pl.ds(0, 16)][0]
        p0a0 = jnp.minimum((s0 // 8) * 8, fp_clamp)
        s_end = ns_v[pl.ds(G, 16)][0]
        lane = lax.iota(jnp.int32, 16)
        negv = jnp.full((16,), -jnp.inf, jnp.float32)
        posv = jnp.full((16,), jnp.inf, jnp.float32)

        def issue(p0, par):
            off = pl.multiple_of(par * 2 * _C, 8)
            q0 = pl.multiple_of(2 * p0, 8)
            pltpu.sync_copy(
                fp2_hbm.at[pl.ds(q0, 2 * _C)], idx_v.at[pl.ds(off, 2 * _C)])
            pltpu.async_copy(
                posi_hbm.at[idx_v.at[pl.ds(off, 2 * _C)]],
                v_v.at[pl.ds(off, 2 * _C)], sem)

        def wait(par):
            off = pl.multiple_of(par * 2 * _C, 8)
            pltpu.make_async_copy(
                posi_hbm.at[idx_v.at[pl.ds(off, 2 * _C)]],
                v_v.at[pl.ds(off, 2 * _C)], sem).wait()

        issue(p0a0, 0)
        wait(0)

        @pl.when(p0a0 + _C < s_end)
        def _():
            issue(jnp.minimum(p0a0 + _C, fp_clamp), 1)

        n_steps = G // 16 + (s_end - p0a0 + _C - 1) // _C + 2

        def step(_, st):
            t, n_cur, cxmax, cxmin, cymax, cymin = st
            active = n_cur < G
            nc = jnp.minimum(n_cur, G)
            p0 = jnp.minimum(p0a0 + t * _C, fp_clamp)
            chunk_end = p0 + _C
            starts = ns_v[pl.ds(nc, 16)]
            ends = ns_v[pl.ds(nc + 1, 16)]
            deg = ends - starts
            validdeg = (deg >= 2) & (deg <= _IGN)
            lo = jnp.maximum(starts, p0)
            hi = jnp.minimum(ends, chunk_end)
            cnt = jnp.where(validdeg, jnp.maximum(hi - lo, 0), 0)
            base2 = 2 * (lo - p0) + (t & 1) * 2 * _C
            is0 = lane == 0
            axmax0 = jnp.where(is0, cxmax, negv)
            axmin0 = jnp.where(is0, cxmin, posv)
            aymax0 = jnp.where(is0, cymax, negv)
            aymin0 = jnp.where(is0, cymin, posv)
            maxcnt = jnp.max(cnt)
            m0 = cnt > 0
            hi_idx = jnp.minimum(
                base2 + 2 * jnp.maximum(cnt - 1, 0), 4 * _C - 2)

            def j_body(i, acc):
                bxmax, bxmin, bymax, bymin = acc
                kk0 = jnp.minimum(base2 + 4 * i, hi_idx)
                kk1 = jnp.minimum(kk0 + 2, hi_idx)
                x0 = plsc.load_gather(v_v, [kk0])
                y0 = plsc.load_gather(v_v, [kk0 + 1])
                x1 = plsc.load_gather(v_v, [kk1])
                y1 = plsc.load_gather(v_v, [kk1 + 1])
                bxmax = jnp.maximum(bxmax, jnp.maximum(x0, x1))
                bxmin = jnp.minimum(bxmin, jnp.minimum(x0, x1))
                bymax = jnp.maximum(bymax, jnp.maximum(y0, y1))
                bymin = jnp.minimum(bymin, jnp.minimum(y0, y1))
                return (bxmax, bxmin, bymax, bymin)

            axmax, axmin, aymax, aymin = lax.fori_loop(
                0, (maxcnt + 1) // 2, j_body,
                (axmax0, axmin0, aymax0, aymin0))
            axmax = jnp.where(m0, axmax, axmax0)
            axmin = jnp.where(m0, axmin, axmin0)
            aymax = jnp.where(m0, aymax, aymax0)
            aymin = jnp.where(m0, aymin, aymin0)

            finished = ends <= chunk_end
            nfin = plsc.all_reduce_population_count(finished)[0]
            wl = jnp.where(
                validdeg, (axmax - axmin) + (aymax - aymin),
                jnp.zeros((16,), jnp.float32))

            @pl.when(active)
            def _():
                out_v[pl.ds(nc, 16)] = wl

            has_carry = nfin < 16
            perm = jnp.broadcast_to(jnp.minimum(nfin, 15), (16,))
            ncxmax = jnp.where(has_carry, axmax[perm], negv)
            ncxmin = jnp.where(has_carry, axmin[perm], posv)
            ncymax = jnp.where(has_carry, aymax[perm], negv)
            ncymin = jnp.where(has_carry, aymin[perm], posv)

            full = nfin == 16
            new_n = n_cur + jnp.where(active, nfin, 0)
            do_fetch = active & jnp.logical_not(full)
            new_t = jnp.where(do_fetch, t + 1, t)

            @pl.when(do_fetch)
            def _():
                nxt_par = 1 - (t & 1)
                wait(nxt_par)
                np1 = jnp.minimum(p0 + _C, fp_clamp)

                @pl.when(np1 + _C < s_end)
                def _():
                    issue(jnp.minimum(np1 + _C, fp_clamp), t & 1)

            return (new_t, new_n, ncxmax, ncxmin, ncymax, ncymin)

        lax.fori_loop(
            0, n_steps, step,
            (jnp.int32(0), jnp.int32(0), negv, posv, negv, posv))
        pltpu.sync_copy(out_v.at[pl.ds(0, G)], out_hbm.at[pl.ds(nw0, G)])

    return k


def kernel(pos, flat_netpin, netpin_start, read_lut_flag):
    n_pins = flat_netpin.shape[0]
    n_nets = netpin_start.shape[0] - 1
    G = -(-n_nets // _NW)
    G = ((G + 7) // 8) * 8
    SS = ((G + 33 + 7) // 8) * 8
    pad = jnp.full((_NW * G + 64 - (n_nets + 1),), n_pins, jnp.int32)
    ns_pad = jnp.concatenate([netpin_start.astype(jnp.int32), pad])
    pos_i = jnp.stack([pos[:n_pins], pos[n_pins:]], axis=1).reshape(-1)
    fp2 = jnp.stack(
        [2 * flat_netpin, 2 * flat_netpin + 1], axis=1).reshape(-1)
    out = _build(n_nets, n_pins, G, SS)(ns_pad, fp2, pos_i)
    return out[:n_nets]


# speedup vs baseline: 9.3625x; 9.3625x over previous
import functools

import jax
import jax.numpy as jnp
from jax import lax
from jax.experimental import pallas as pl
from jax.experimental.pallas import tpu as pltpu
from jax.experimental.pallas import tpu_sc as plsc

_NW = 32
_C = 2048
_IGN = 100


@functools.lru_cache(maxsize=None)
def _build(n_nets, n_pins, G, SS):
    mesh = plsc.VectorSubcoreMesh(core_axis_name="c", subcore_axis_name="s")
    ninf = jnp.float32(-jnp.inf)
    pinf = jnp.float32(jnp.inf)

    fp_clamp = n_pins - _C
    assert fp_clamp % 8 == 0 and fp_clamp > 0

    @functools.partial(
        pl.kernel,
        mesh=mesh,
        out_type=jax.ShapeDtypeStruct((_NW * G,), jnp.float32),
        scratch_types=[
            pltpu.VMEM((SS,), jnp.int32),
            pltpu.VMEM((2 * _C,), jnp.int32),
            pltpu.VMEM((2 * _C,), jnp.float32),
            pltpu.VMEM((2 * _C,), jnp.float32),
            pltpu.VMEM((G + 16,), jnp.float32),
            pltpu.SemaphoreType.DMA,
            pltpu.SemaphoreType.DMA,
        ],
        compiler_params=pltpu.CompilerParams(needs_layout_passes=False),
    )
    def k(ns_hbm, fp_hbm, pos_hbm, out_hbm,
          ns_v, idx_v, x_v, y_v, out_v, semx, semy):
        wid = lax.axis_index("s") * 2 + lax.axis_index("c")
        nw0 = wid * G
        pltpu.sync_copy(ns_hbm.at[pl.ds(nw0, SS)], ns_v)
        s0 = ns_v[pl.ds(0, 16)][0]
        p0a0 = jnp.minimum((s0 // 8) * 8, fp_clamp)
        s_end = ns_v[pl.ds(G, 16)][0]
        lane = lax.iota(jnp.int32, 16)
        negv = jnp.full((16,), -jnp.inf, jnp.float32)
        posv = jnp.full((16,), jnp.inf, jnp.float32)
        ys_hbm = pos_hbm.at[pl.ds(n_pins, n_pins)]

        def issue(p0, par):
            off = pl.multiple_of(par * _C, 8)
            p0 = pl.multiple_of(p0, 8)
            pltpu.sync_copy(fp_hbm.at[pl.ds(p0, _C)], idx_v.at[pl.ds(off, _C)])
            pltpu.async_copy(
                pos_hbm.at[idx_v.at[pl.ds(off, _C)]], x_v.at[pl.ds(off, _C)],
                semx)
            pltpu.async_copy(
                ys_hbm.at[idx_v.at[pl.ds(off, _C)]], y_v.at[pl.ds(off, _C)],
                semy)

        def wait(par):
            off = pl.multiple_of(par * _C, 8)
            pltpu.make_async_copy(
                pos_hbm.at[idx_v.at[pl.ds(off, _C)]], x_v.at[pl.ds(off, _C)],
                semx).wait()
            pltpu.make_async_copy(
                ys_hbm.at[idx_v.at[pl.ds(off, _C)]], y_v.at[pl.ds(off, _C)],
                semy).wait()

        issue(p0a0, 0)
        wait(0)

        @pl.when(p0a0 + _C < s_end)
        def _():
            issue(jnp.minimum(p0a0 + _C, fp_clamp), 1)

        n_steps = G // 16 + (s_end - p0a0 + _C - 1) // _C + 2

        def step(_, st):
            t, n_cur, cxmax, cxmin, cymax, cymin = st
            active = n_cur < G
            nc = jnp.minimum(n_cur, G)
            p0 = jnp.minimum(p0a0 + t * _C, fp_clamp)
            chunk_end = p0 + _C
            starts = ns_v[pl.ds(nc, 16)]
            ends = ns_v[pl.ds(nc + 1, 16)]
            deg = ends - starts
            validdeg = (deg >= 2) & (deg <= _IGN)
            lo = jnp.maximum(starts, p0)
            hi = jnp.minimum(ends, chunk_end)
            cnt = jnp.where(validdeg, jnp.maximum(hi - lo, 0), 0)
            paroff = (t & 1) * _C
            base_k = lo - p0 + paroff
            is0 = lane == 0
            axmax0 = jnp.where(is0, cxmax, negv)
            axmin0 = jnp.where(is0, cxmin, posv)
            aymax0 = jnp.where(is0, cymax, negv)
            aymin0 = jnp.where(is0, cymin, posv)
            maxcnt = jnp.max(cnt)
            m0 = cnt > 0
            hi_idx = jnp.minimum(
                base_k + jnp.maximum(cnt - 1, 0), 2 * _C - 1)

            def j_body(i, acc):
                bxmax, bxmin, bymax, bymin = acc
                kk0 = jnp.minimum(base_k + 2 * i, hi_idx)
                kk1 = jnp.minimum(kk0 + 1, hi_idx)
                x0 = plsc.load_gather(x_v, [kk0])
                y0 = plsc.load_gather(y_v, [kk0])
                x1 = plsc.load_gather(x_v, [kk1])
                y1 = plsc.load_gather(y_v, [kk1])
                bxmax = jnp.maximum(bxmax, jnp.maximum(x0, x1))
                bxmin = jnp.minimum(bxmin, jnp.minimum(x0, x1))
                bymax = jnp.maximum(bymax, jnp.maximum(y0, y1))
                bymin = jnp.minimum(bymin, jnp.minimum(y0, y1))
                return (bxmax, bxmin, bymax, bymin)

            axmax, axmin, aymax, aymin = lax.fori_loop(
                0, (maxcnt + 1) // 2, j_body,
                (axmax0, axmin0, aymax0, aymin0))
            axmax = jnp.where(m0, axmax, axmax0)
            axmin = jnp.where(m0, axmin, axmin0)
            aymax = jnp.where(m0, aymax, aymax0)
            aymin = jnp.where(m0, aymin, aymin0)

            finished = ends <= chunk_end
            nfin = plsc.all_reduce_population_count(finished)[0]
            wl = jnp.where(
                validdeg, (axmax - axmin) + (aymax - aymin),
                jnp.zeros((16,), jnp.float32))

            @pl.when(active)
            def _():
                out_v[pl.ds(nc, 16)] = wl

            has_carry = nfin < 16
            perm = jnp.broadcast_to(jnp.minimum(nfin, 15), (16,))
            ncxmax = jnp.where(has_carry, axmax[perm], negv)
            ncxmin = jnp.where(has_carry, axmin[perm], posv)
            ncymax = jnp.where(has_carry, aymax[perm], negv)
            ncymin = jnp.where(has_carry, aymin[perm], posv)

            full = nfin == 16
            new_n = n_cur + jnp.where(active, nfin, 0)
            do_fetch = active & jnp.logical_not(full)
            new_t = jnp.where(do_fetch, t + 1, t)

            @pl.when(do_fetch)
            def _():
                nxt_par = 1 - (t & 1)
                wait(nxt_par)
                np1 = jnp.minimum(p0 + _C, fp_clamp)

                @pl.when(np1 + _C < s_end)
                def _():
                    issue(jnp.minimum(np1 + _C, fp_clamp), t & 1)

            return (new_t, new_n, ncxmax, ncxmin, ncymax, ncymin)

        lax.fori_loop(
            0, n_steps, step,
            (jnp.int32(0), jnp.int32(0), negv, posv, negv, posv))
        pltpu.sync_copy(out_v.at[pl.ds(0, G)], out_hbm.at[pl.ds(nw0, G)])

    return k


def kernel(pos, flat_netpin, netpin_start, read_lut_flag):
    n_pins = flat_netpin.shape[0]
    n_nets = netpin_start.shape[0] - 1
    G = -(-n_nets // _NW)
    G = ((G + 7) // 8) * 8
    SS = ((G + 33 + 7) // 8) * 8
    pad = jnp.full((_NW * G + 64 - (n_nets + 1),), n_pins, jnp.int32)
    ns_pad = jnp.concatenate([netpin_start.astype(jnp.int32), pad])
    out = _build(n_nets, n_pins, G, SS)(ns_pad, flat_netpin, pos)
    return out[:n_nets]
